# baseline (speedup 1.0000x reference)
import functools

import jax
import jax.numpy as jnp
from jax import lax
from jax.experimental import pallas as pl
from jax.experimental.pallas import tpu as pltpu
from jax.experimental.pallas import tpu_sc as plsc

_L = 128
_NW = 32
_NTILES = 16
_NBUF = 3


def _make_sc_agg(N, D, CH, NPAD):
    FULL = ((N + _NTILES - 1) // _NTILES + 7) // 8 * 8
    LAST = N - (_NTILES - 1) * FULL
    mesh = plsc.VectorSubcoreMesh(core_axis_name="c", subcore_axis_name="s")

    @functools.partial(
        pl.kernel, mesh=mesh,
        out_type=jax.ShapeDtypeStruct((2, N, D), jnp.float32),
        scratch_types=[
            pltpu.VMEM((_NBUF, 2, _L), jnp.int32),
            pltpu.VMEM((_NBUF, 2, _L), jnp.int32),
        ] + [pltpu.VMEM((_L, D), jnp.float32) for _ in range(_NBUF)]
        + [pltpu.VMEM_SHARED((NPAD, D), jnp.float32)]
        + [pltpu.SemaphoreType.DMA for _ in range(3 * _NBUF)],
    )
    def sc_agg(h_hbm, src_hbm, dst_hbm, out_hbm, src_i, dst_i, *rest):
        bufs = rest[:_NBUF]
        acc_sh = rest[_NBUF]
        isems = rest[_NBUF + 1:2 * _NBUF + 1]
        gsems = rest[2 * _NBUF + 1:3 * _NBUF + 1]
        ssems = rest[3 * _NBUF + 1:]
        c = lax.axis_index("c")
        s = lax.axis_index("s")
        w = c * _NTILES + s
        e0 = w * (CH * _L)

        def issue_idx(b, p, j):
            pltpu.async_copy(src_hbm.at[pl.ds(e0 + j * _L, _L)],
                             src_i.at[b, p], isems[b])
            pltpu.async_copy(dst_hbm.at[pl.ds(e0 + j * _L, _L)],
                             dst_i.at[b, p], isems[b])

        def wait_idx(b, p, j):
            pltpu.make_async_copy(src_hbm.at[pl.ds(e0 + j * _L, _L)],
                                  src_i.at[b, p], isems[b]).wait()
            pltpu.make_async_copy(dst_hbm.at[pl.ds(e0 + j * _L, _L)],
                                  dst_i.at[b, p], isems[b]).wait()

        r0 = pl.multiple_of(s * FULL, 8)

        for b in range(_NBUF):
            issue_idx(b, 0, b)

        @pl.when(s < _NTILES - 1)
        def _():
            pltpu.sync_copy(h_hbm.at[pl.ds(r0, FULL)],
                            acc_sh.at[pl.ds(r0, FULL)])

        @pl.when(s == _NTILES - 1)
        def _():
            pltpu.sync_copy(h_hbm.at[pl.ds(r0, LAST)],
                            acc_sh.at[pl.ds(r0, LAST)])

        plsc.subcore_barrier()

        def body(i, carry):
            j0 = i * _NBUF
            p = i % 2
            for b in range(_NBUF):
                j = j0 + b

                @pl.when(i > 0)
                def _(b=b):
                    pltpu.make_async_copy(
                        bufs[b], acc_sh.at[dst_i.at[b, 1 - p]],
                        ssems[b]).wait()

                wait_idx(b, p, j)
                pltpu.async_copy(h_hbm.at[src_i.at[b, p]], bufs[b], gsems[b])

                @pl.when(j + _NBUF < CH)
                def _(b=b, j=j):
                    issue_idx(b, 1 - p, j + _NBUF)
            for b in range(_NBUF):
                pltpu.make_async_copy(h_hbm.at[src_i.at[b, p]], bufs[b],
                                      gsems[b]).wait()
                pltpu.async_copy(bufs[b], acc_sh.at[dst_i.at[b, p]], ssems[b],
                                 add=True)
            return carry

        lax.fori_loop(0, CH // _NBUF, body, 0)
        pfin = (CH // _NBUF - 1) % 2
        for b in range(_NBUF):
            pltpu.make_async_copy(bufs[b], acc_sh.at[dst_i.at[b, pfin]],
                                  ssems[b]).wait()
        plsc.subcore_barrier()

        @pl.when(s < _NTILES - 1)
        def _():
            pltpu.sync_copy(acc_sh.at[pl.ds(r0, FULL)],
                            out_hbm.at[c, pl.ds(r0, FULL)])

        @pl.when(s == _NTILES - 1)
        def _():
            pltpu.sync_copy(acc_sh.at[pl.ds(r0, LAST)],
                            out_hbm.at[c, pl.ds(r0, LAST)])

    return sc_agg


def _mlp_body(G, N, h_ref, a0_ref, a1_ref, b_ref, w1_ref, b1_ref, g1_ref,
              be1_ref, w2_ref, b2_ref, g2_ref, be2_ref, hout_ref, pool_ref):
    y = a0_ref[...] + a1_ref[...] - h_ref[...]
    h1 = jnp.maximum(
        jnp.dot(y, w1_ref[...], preferred_element_type=jnp.float32)
        + b1_ref[...], 0.0)
    m1 = jnp.mean(h1, axis=0, keepdims=True)
    v1 = jnp.mean((h1 - m1) ** 2, axis=0, keepdims=True)
    h1 = (h1 - m1) * lax.rsqrt(v1 + 1e-5) * g1_ref[...] + be1_ref[...]
    h2 = jnp.maximum(
        jnp.dot(h1, w2_ref[...], preferred_element_type=jnp.float32)
        + b2_ref[...], 0.0)
    m2 = jnp.mean(h2, axis=0, keepdims=True)
    v2 = jnp.mean((h2 - m2) ** 2, axis=0, keepdims=True)
    h2 = (h2 - m2) * lax.rsqrt(v2 + 1e-5) * g2_ref[...] + be2_ref[...]
    hout_ref[...] = h2
    gids = lax.broadcasted_iota(jnp.int32, (G, N), 0)
    onehot = (gids == b_ref[...]).astype(jnp.float32)
    pool_ref[...] = jnp.dot(onehot, h2, preferred_element_type=jnp.float32)


def _cls_body(f_ref, gc_ref, bcn_ref, w1_ref, b1_ref, w2_ref, b2_ref, o_ref):
    f = f_ref[...]
    m = jnp.mean(f, axis=0, keepdims=True)
    v = jnp.mean((f - m) ** 2, axis=0, keepdims=True)
    f = (f - m) * lax.rsqrt(v + 1e-5) * gc_ref[...] + bcn_ref[...]
    f = jnp.maximum(
        jnp.dot(f, w1_ref[...], preferred_element_type=jnp.float32)
        + b1_ref[...], 0.0)
    z = jnp.dot(f, w2_ref[...], preferred_element_type=jnp.float32) + b2_ref[...]
    zm = jnp.max(z, axis=-1, keepdims=True)
    o_ref[...] = (z - zm) - jnp.log(
        jnp.sum(jnp.exp(z - zm), axis=-1, keepdims=True))


def kernel(x, edge_index, batch, W1, b1, g1, be1, W2, b2, g2, be2, gc, bcn,
           Wc1, bc1, Wc2, bc2):
    N, D = x.shape
    E = edge_index.shape[1]
    BLOCKS = W1.shape[0]
    G = 64
    C = Wc2.shape[1]

    CH = -(-E // (_NW * _L))
    CH = -(-CH // _NBUF) * _NBUF
    Epad = _NW * CH * _L
    NPAD = N + 8
    src = edge_index[0]
    dst = edge_index[1]
    pad = Epad - E
    srcp = jnp.concatenate([src, jnp.zeros((pad,), jnp.int32)])
    dstp = jnp.concatenate([dst, jnp.full((pad,), N, jnp.int32)])
    batch_row = batch.reshape(1, N)

    sc_agg = _make_sc_agg(N, D, CH, NPAD)

    mlp = pl.pallas_call(
        functools.partial(_mlp_body, G, N),
        out_shape=[
            jax.ShapeDtypeStruct((N, D), jnp.float32),
            jax.ShapeDtypeStruct((G, D), jnp.float32),
        ],
    )

    cls = pl.pallas_call(
        _cls_body,
        out_shape=jax.ShapeDtypeStruct((G, C), jnp.float32),
    )

    h = x
    pooled = []
    for i in range(BLOCKS):
        acc = sc_agg(h, srcp, dstp)
        h, pool_i = mlp(h, acc[0], acc[1], batch_row,
                        W1[i], b1[i].reshape(1, D), g1[i].reshape(1, D),
                        be1[i].reshape(1, D),
                        W2[i], b2[i].reshape(1, D), g2[i].reshape(1, D),
                        be2[i].reshape(1, D))
        pooled.append(pool_i)

    f = jnp.concatenate(pooled, axis=1)
    return cls(f, gc.reshape(1, -1), bcn.reshape(1, -1), Wc1,
               bc1.reshape(1, -1), Wc2, bc2.reshape(1, -1))


# speedup vs baseline: 1.2874x; 1.2874x over previous
import functools

import jax
import jax.numpy as jnp
from jax import lax
from jax.experimental import pallas as pl
from jax.experimental.pallas import tpu as pltpu
from jax.experimental.pallas import tpu_sc as plsc

_L = 128
_NW = 32
_NTILES = 16
_NBUF = 3


def _make_sc_agg(N, D, CH, NPAD):
    FULL = ((N + _NTILES - 1) // _NTILES + 7) // 8 * 8
    LAST = N - (_NTILES - 1) * FULL
    mesh = plsc.VectorSubcoreMesh(core_axis_name="c", subcore_axis_name="s")

    CH2 = CH // 2

    @functools.partial(
        pl.kernel, mesh=mesh,
        out_type=jax.ShapeDtypeStruct((2, N, D), jnp.float32),
        scratch_types=[
            pltpu.VMEM((CH2, _L), jnp.int32),
            pltpu.VMEM((CH2, _L), jnp.int32),
            pltpu.VMEM((_L, D), jnp.float32),
            pltpu.VMEM((_L, D), jnp.float32),
            pltpu.VMEM_SHARED((NPAD, D), jnp.float32),
            pltpu.SemaphoreType.DMA,
            pltpu.SemaphoreType.DMA,
            pltpu.SemaphoreType.DMA,
            pltpu.SemaphoreType.DMA,
        ],
    )
    def sc_agg(h_hbm, src_hbm, dst_hbm, out_hbm, src_v, dst_v, bufA, bufB,
               acc_sh, gA, gB, sA, sB):
        c = lax.axis_index("c")
        s = lax.axis_index("s")
        w = c * _NTILES + s
        r0 = pl.multiple_of(s * FULL, 8)

        @pl.when(s < _NTILES - 1)
        def _():
            pltpu.sync_copy(h_hbm.at[pl.ds(r0, FULL)],
                            acc_sh.at[pl.ds(r0, FULL)])

        @pl.when(s == _NTILES - 1)
        def _():
            pltpu.sync_copy(h_hbm.at[pl.ds(r0, LAST)],
                            acc_sh.at[pl.ds(r0, LAST)])

        plsc.subcore_barrier()

        for half in range(2):
            pltpu.sync_copy(src_hbm.at[pl.ds(w * CH + half * CH2, CH2)],
                            src_v)
            pltpu.sync_copy(dst_hbm.at[pl.ds(w * CH + half * CH2, CH2)],
                            dst_v)
            pltpu.async_copy(h_hbm.at[src_v.at[0]], bufA, gA)

            def body(i, carry):
                j0 = i * 2
                pltpu.make_async_copy(h_hbm.at[src_v.at[j0]], bufA, gA).wait()
                pltpu.async_copy(bufA, acc_sh.at[dst_v.at[j0]], sA, add=True)

                @pl.when(i > 0)
                def _():
                    pltpu.make_async_copy(bufB, acc_sh.at[dst_v.at[j0 - 1]],
                                          sB).wait()

                pltpu.async_copy(h_hbm.at[src_v.at[j0 + 1]], bufB, gB)
                pltpu.make_async_copy(h_hbm.at[src_v.at[j0 + 1]], bufB,
                                      gB).wait()
                pltpu.async_copy(bufB, acc_sh.at[dst_v.at[j0 + 1]], sB,
                                 add=True)

                @pl.when(j0 + 2 < CH2)
                def _():
                    pltpu.make_async_copy(bufA, acc_sh.at[dst_v.at[j0]],
                                          sA).wait()
                    pltpu.async_copy(h_hbm.at[src_v.at[j0 + 2]], bufA, gA)

                return carry

            lax.fori_loop(0, CH2 // 2, body, 0)
            pltpu.make_async_copy(bufA, acc_sh.at[dst_v.at[CH2 - 2]],
                                  sA).wait()
            pltpu.make_async_copy(bufB, acc_sh.at[dst_v.at[CH2 - 1]],
                                  sB).wait()

        plsc.subcore_barrier()

        @pl.when(s < _NTILES - 1)
        def _():
            pltpu.sync_copy(acc_sh.at[pl.ds(r0, FULL)],
                            out_hbm.at[c, pl.ds(r0, FULL)])

        @pl.when(s == _NTILES - 1)
        def _():
            pltpu.sync_copy(acc_sh.at[pl.ds(r0, LAST)],
                            out_hbm.at[c, pl.ds(r0, LAST)])

    return sc_agg


def _mlp_body(G, N, h_ref, a0_ref, a1_ref, b_ref, w1_ref, b1_ref, g1_ref,
              be1_ref, w2_ref, b2_ref, g2_ref, be2_ref, hout_ref, pool_ref):
    y = a0_ref[...] + a1_ref[...] - h_ref[...]
    h1 = jnp.maximum(
        jnp.dot(y, w1_ref[...], preferred_element_type=jnp.float32)
        + b1_ref[...], 0.0)
    m1 = jnp.mean(h1, axis=0, keepdims=True)
    v1 = jnp.mean((h1 - m1) ** 2, axis=0, keepdims=True)
    h1 = (h1 - m1) * lax.rsqrt(v1 + 1e-5) * g1_ref[...] + be1_ref[...]
    h2 = jnp.maximum(
        jnp.dot(h1, w2_ref[...], preferred_element_type=jnp.float32)
        + b2_ref[...], 0.0)
    m2 = jnp.mean(h2, axis=0, keepdims=True)
    v2 = jnp.mean((h2 - m2) ** 2, axis=0, keepdims=True)
    h2 = (h2 - m2) * lax.rsqrt(v2 + 1e-5) * g2_ref[...] + be2_ref[...]
    hout_ref[...] = h2
    gids = lax.broadcasted_iota(jnp.int32, (G, N), 0)
    onehot = (gids == b_ref[...]).astype(jnp.float32)
    pool_ref[...] = jnp.dot(onehot, h2, preferred_element_type=jnp.float32)


def _cls_body(f_ref, gc_ref, bcn_ref, w1_ref, b1_ref, w2_ref, b2_ref, o_ref):
    f = f_ref[...]
    m = jnp.mean(f, axis=0, keepdims=True)
    v = jnp.mean((f - m) ** 2, axis=0, keepdims=True)
    f = (f - m) * lax.rsqrt(v + 1e-5) * gc_ref[...] + bcn_ref[...]
    f = jnp.maximum(
        jnp.dot(f, w1_ref[...], preferred_element_type=jnp.float32)
        + b1_ref[...], 0.0)
    z = jnp.dot(f, w2_ref[...], preferred_element_type=jnp.float32) + b2_ref[...]
    zm = jnp.max(z, axis=-1, keepdims=True)
    o_ref[...] = (z - zm) - jnp.log(
        jnp.sum(jnp.exp(z - zm), axis=-1, keepdims=True))


def kernel(x, edge_index, batch, W1, b1, g1, be1, W2, b2, g2, be2, gc, bcn,
           Wc1, bc1, Wc2, bc2):
    N, D = x.shape
    E = edge_index.shape[1]
    BLOCKS = W1.shape[0]
    G = 64
    C = Wc2.shape[1]

    CH = -(-E // (_NW * _L))
    CH = -(-CH // 4) * 4
    Epad = _NW * CH * _L
    NPAD = N + 8
    src = edge_index[0]
    dst = edge_index[1]
    pad = Epad - E
    srcp = jnp.concatenate(
        [src, jnp.zeros((pad,), jnp.int32)]).reshape(_NW * CH, _L)
    dstp = jnp.concatenate(
        [dst, jnp.full((pad,), N, jnp.int32)]).reshape(_NW * CH, _L)
    batch_row = batch.reshape(1, N)

    sc_agg = _make_sc_agg(N, D, CH, NPAD)

    mlp = pl.pallas_call(
        functools.partial(_mlp_body, G, N),
        out_shape=[
            jax.ShapeDtypeStruct((N, D), jnp.float32),
            jax.ShapeDtypeStruct((G, D), jnp.float32),
        ],
    )

    cls = pl.pallas_call(
        _cls_body,
        out_shape=jax.ShapeDtypeStruct((G, C), jnp.float32),
    )

    h = x
    pooled = []
    for i in range(BLOCKS):
        acc = sc_agg(h, srcp, dstp)
        h, pool_i = mlp(h, acc[0], acc[1], batch_row,
                        W1[i], b1[i].reshape(1, D), g1[i].reshape(1, D),
                        be1[i].reshape(1, D),
                        W2[i], b2[i].reshape(1, D), g2[i].reshape(1, D),
                        be2[i].reshape(1, D))
        pooled.append(pool_i)

    f = jnp.concatenate(pooled, axis=1)
    return cls(f, gc.reshape(1, -1), bcn.reshape(1, -1), Wc1,
               bc1.reshape(1, -1), Wc2, bc2.reshape(1, -1))


# speedup vs baseline: 3.7185x; 2.8885x over previous
import functools

import jax
import jax.numpy as jnp
from jax import lax
from jax.experimental import pallas as pl
from jax.experimental.pallas import tpu as pltpu
from jax.experimental.pallas import tpu_sc as plsc

_L = 128
_NW = 32
_NTILES = 16
_NBUF = 3


def _make_sc_agg(N, D, CH, NPAD):
    FULL = ((N + _NTILES - 1) // _NTILES + 7) // 8 * 8
    LAST = N - (_NTILES - 1) * FULL
    mesh = plsc.VectorSubcoreMesh(core_axis_name="c", subcore_axis_name="s")

    CH2 = CH // 2

    @functools.partial(
        pl.kernel, mesh=mesh,
        out_type=jax.ShapeDtypeStruct((2, N, D), jnp.float32),
        scratch_types=[
            pltpu.VMEM((CH2, _L), jnp.int32),
            pltpu.VMEM((CH2, _L), jnp.int32),
            pltpu.VMEM((_L, D), jnp.float32),
            pltpu.VMEM((_L, D), jnp.float32),
            pltpu.VMEM_SHARED((NPAD, D), jnp.float32),
            pltpu.SemaphoreType.DMA,
            pltpu.SemaphoreType.DMA,
            pltpu.SemaphoreType.DMA,
            pltpu.SemaphoreType.DMA,
        ],
    )
    def sc_agg(h_hbm, src_hbm, dst_hbm, out_hbm, src_v, dst_v, bufA, bufB,
               acc_sh, gA, gB, sA, sB):
        c = lax.axis_index("c")
        s = lax.axis_index("s")
        w = c * _NTILES + s
        r0 = pl.multiple_of(s * FULL, 8)

        @pl.when(s < _NTILES - 1)
        def _():
            pltpu.sync_copy(h_hbm.at[pl.ds(r0, FULL)],
                            acc_sh.at[pl.ds(r0, FULL)])

        @pl.when(s == _NTILES - 1)
        def _():
            pltpu.sync_copy(h_hbm.at[pl.ds(r0, LAST)],
                            acc_sh.at[pl.ds(r0, LAST)])

        plsc.subcore_barrier()

        for half in range(2):
            pltpu.sync_copy(src_hbm.at[pl.ds(w * CH + half * CH2, CH2)],
                            src_v)
            pltpu.sync_copy(dst_hbm.at[pl.ds(w * CH + half * CH2, CH2)],
                            dst_v)
            pltpu.async_copy(h_hbm.at[src_v.at[0]], bufA, gA)

            def body(i, carry):
                j0 = i * 2
                pltpu.make_async_copy(h_hbm.at[src_v.at[j0]], bufA, gA).wait()
                pltpu.async_copy(bufA, acc_sh.at[dst_v.at[j0]], sA, add=True)

                @pl.when(i > 0)
                def _():
                    pltpu.make_async_copy(bufB, acc_sh.at[dst_v.at[j0 - 1]],
                                          sB).wait()

                pltpu.async_copy(h_hbm.at[src_v.at[j0 + 1]], bufB, gB)
                pltpu.make_async_copy(h_hbm.at[src_v.at[j0 + 1]], bufB,
                                      gB).wait()
                pltpu.async_copy(bufB, acc_sh.at[dst_v.at[j0 + 1]], sB,
                                 add=True)

                @pl.when(j0 + 2 < CH2)
                def _():
                    pltpu.make_async_copy(bufA, acc_sh.at[dst_v.at[j0]],
                                          sA).wait()
                    pltpu.async_copy(h_hbm.at[src_v.at[j0 + 2]], bufA, gA)

                return carry

            lax.fori_loop(0, CH2 // 2, body, 0)
            pltpu.make_async_copy(bufA, acc_sh.at[dst_v.at[CH2 - 2]],
                                  sA).wait()
            pltpu.make_async_copy(bufB, acc_sh.at[dst_v.at[CH2 - 1]],
                                  sB).wait()

        plsc.subcore_barrier()

        @pl.when(s < _NTILES - 1)
        def _():
            pltpu.sync_copy(acc_sh.at[pl.ds(r0, FULL)],
                            out_hbm.at[c, pl.ds(r0, FULL)])

        @pl.when(s == _NTILES - 1)
        def _():
            pltpu.sync_copy(acc_sh.at[pl.ds(r0, LAST)],
                            out_hbm.at[c, pl.ds(r0, LAST)])

    return sc_agg


def _mlp_body(G, N, h_ref, a0_ref, a1_ref, b_ref, w1_ref, b1_ref, g1_ref,
              be1_ref, w2_ref, b2_ref, g2_ref, be2_ref, hout_ref, pool_ref):
    y = a0_ref[...] + a1_ref[...] - h_ref[...]
    h1 = jnp.maximum(
        jnp.dot(y, w1_ref[...], preferred_element_type=jnp.float32, precision=lax.Precision.HIGHEST)
        + b1_ref[...], 0.0)
    m1 = jnp.mean(h1, axis=0, keepdims=True)
    v1 = jnp.mean((h1 - m1) ** 2, axis=0, keepdims=True)
    h1 = (h1 - m1) / jnp.sqrt(v1 + 1e-5) * g1_ref[...] + be1_ref[...]
    h2 = jnp.maximum(
        jnp.dot(h1, w2_ref[...], preferred_element_type=jnp.float32, precision=lax.Precision.HIGHEST)
        + b2_ref[...], 0.0)
    m2 = jnp.mean(h2, axis=0, keepdims=True)
    v2 = jnp.mean((h2 - m2) ** 2, axis=0, keepdims=True)
    h2 = (h2 - m2) / jnp.sqrt(v2 + 1e-5) * g2_ref[...] + be2_ref[...]
    hout_ref[...] = h2
    gids = lax.broadcasted_iota(jnp.int32, (G, N), 0)
    onehot = (gids == b_ref[...]).astype(jnp.float32)
    pool_ref[...] = jnp.dot(onehot, h2, preferred_element_type=jnp.float32, precision=lax.Precision.HIGHEST)


def _cls_body(f_ref, gc_ref, bcn_ref, w1_ref, b1_ref, w2_ref, b2_ref, o_ref):
    f = f_ref[...]
    m = jnp.mean(f, axis=0, keepdims=True)
    v = jnp.mean((f - m) ** 2, axis=0, keepdims=True)
    f = (f - m) / jnp.sqrt(v + 1e-5) * gc_ref[...] + bcn_ref[...]
    f = jnp.maximum(
        jnp.dot(f, w1_ref[...], preferred_element_type=jnp.float32, precision=lax.Precision.HIGHEST)
        + b1_ref[...], 0.0)
    z = jnp.dot(f, w2_ref[...], preferred_element_type=jnp.float32, precision=lax.Precision.HIGHEST) + b2_ref[...]
    zm = jnp.max(z, axis=-1, keepdims=True)
    o_ref[...] = (z - zm) - jnp.log(
        jnp.sum(jnp.exp(z - zm), axis=-1, keepdims=True))


def kernel(x, edge_index, batch, W1, b1, g1, be1, W2, b2, g2, be2, gc, bcn,
           Wc1, bc1, Wc2, bc2):
    N, D = x.shape
    E = edge_index.shape[1]
    BLOCKS = W1.shape[0]
    G = 64
    C = Wc2.shape[1]

    CH = -(-E // (_NW * _L))
    CH = -(-CH // 4) * 4
    Epad = _NW * CH * _L
    NPAD = N + _L
    src = edge_index[0]
    dst = edge_index[1]
    pad = Epad - E
    lanes = jnp.arange(pad, dtype=jnp.int32) % _L
    srcp = jnp.concatenate([src, lanes]).reshape(_NW * CH, _L)
    dstp = jnp.concatenate([dst, N + lanes]).reshape(_NW * CH, _L)
    batch_row = batch.reshape(1, N)

    sc_agg = _make_sc_agg(N, D, CH, NPAD)

    mlp = pl.pallas_call(
        functools.partial(_mlp_body, G, N),
        out_shape=[
            jax.ShapeDtypeStruct((N, D), jnp.float32),
            jax.ShapeDtypeStruct((G, D), jnp.float32),
        ],
    )

    cls = pl.pallas_call(
        _cls_body,
        out_shape=jax.ShapeDtypeStruct((G, C), jnp.float32),
    )

    h = x
    pooled = []
    for i in range(BLOCKS):
        acc = sc_agg(h, srcp, dstp)
        h, pool_i = mlp(h, acc[0], acc[1], batch_row,
                        W1[i], b1[i].reshape(1, D), g1[i].reshape(1, D),
                        be1[i].reshape(1, D),
                        W2[i], b2[i].reshape(1, D), g2[i].reshape(1, D),
                        be2[i].reshape(1, D))
        pooled.append(pool_i)

    f = jnp.concatenate(pooled, axis=1)
    return cls(f, gc.reshape(1, -1), bcn.reshape(1, -1), Wc1,
               bc1.reshape(1, -1), Wc2, bc2.reshape(1, -1))


# speedup vs baseline: 3.7298x; 1.0030x over previous
import functools

import jax
import jax.numpy as jnp
from jax import lax
from jax.experimental import pallas as pl
from jax.experimental.pallas import tpu as pltpu
from jax.experimental.pallas import tpu_sc as plsc

_L = 128
_NW = 32
_NTILES = 16
_NBUF = 3


def _make_sc_agg(N, D, CH, NPAD):
    FULL = ((N + _NTILES - 1) // _NTILES + 7) // 8 * 8
    LAST = N - (_NTILES - 1) * FULL
    mesh = plsc.VectorSubcoreMesh(core_axis_name="c", subcore_axis_name="s")

    CH2 = CH // 2

    @functools.partial(
        pl.kernel, mesh=mesh,
        out_type=jax.ShapeDtypeStruct((2, N, D), jnp.float32),
        scratch_types=[
            pltpu.VMEM((CH2, _L), jnp.int32),
            pltpu.VMEM((CH2, _L), jnp.int32),
            pltpu.VMEM((_L, D), jnp.float32),
            pltpu.VMEM((_L, D), jnp.float32),
            pltpu.VMEM_SHARED((NPAD, D), jnp.float32),
            pltpu.SemaphoreType.DMA,
            pltpu.SemaphoreType.DMA,
            pltpu.SemaphoreType.DMA,
            pltpu.SemaphoreType.DMA,
        ],
    )
    def sc_agg(h_hbm, src_hbm, dst_hbm, out_hbm, src_v, dst_v, bufA, bufB,
               acc_sh, gA, gB, sA, sB):
        c = lax.axis_index("c")
        s = lax.axis_index("s")
        w = c * _NTILES + s
        r0 = pl.multiple_of(s * FULL, 8)

        @pl.when(s < _NTILES - 1)
        def _():
            pltpu.sync_copy(h_hbm.at[pl.ds(r0, FULL)],
                            acc_sh.at[pl.ds(r0, FULL)])

        @pl.when(s == _NTILES - 1)
        def _():
            pltpu.sync_copy(h_hbm.at[pl.ds(r0, LAST)],
                            acc_sh.at[pl.ds(r0, LAST)])

        plsc.subcore_barrier()

        for half in range(2):
            pltpu.sync_copy(src_hbm.at[pl.ds(w * CH + half * CH2, CH2)],
                            src_v)
            pltpu.sync_copy(dst_hbm.at[pl.ds(w * CH + half * CH2, CH2)],
                            dst_v)
            pltpu.async_copy(h_hbm.at[src_v.at[0]], bufA, gA)

            def body(i, carry):
                j0 = i * 2
                pltpu.make_async_copy(h_hbm.at[src_v.at[j0]], bufA, gA).wait()
                pltpu.async_copy(bufA, acc_sh.at[dst_v.at[j0]], sA, add=True)

                @pl.when(i > 0)
                def _():
                    pltpu.make_async_copy(bufB, acc_sh.at[dst_v.at[j0 - 1]],
                                          sB).wait()

                pltpu.async_copy(h_hbm.at[src_v.at[j0 + 1]], bufB, gB)
                pltpu.make_async_copy(h_hbm.at[src_v.at[j0 + 1]], bufB,
                                      gB).wait()
                pltpu.async_copy(bufB, acc_sh.at[dst_v.at[j0 + 1]], sB,
                                 add=True)

                @pl.when(j0 + 2 < CH2)
                def _():
                    pltpu.make_async_copy(bufA, acc_sh.at[dst_v.at[j0]],
                                          sA).wait()
                    pltpu.async_copy(h_hbm.at[src_v.at[j0 + 2]], bufA, gA)

                return carry

            lax.fori_loop(0, CH2 // 2, body, 0)
            pltpu.make_async_copy(bufA, acc_sh.at[dst_v.at[CH2 - 2]],
                                  sA).wait()
            pltpu.make_async_copy(bufB, acc_sh.at[dst_v.at[CH2 - 1]],
                                  sB).wait()

        plsc.subcore_barrier()

        @pl.when(s < _NTILES - 1)
        def _():
            pltpu.sync_copy(acc_sh.at[pl.ds(r0, FULL)],
                            out_hbm.at[c, pl.ds(r0, FULL)])

        @pl.when(s == _NTILES - 1)
        def _():
            pltpu.sync_copy(acc_sh.at[pl.ds(r0, LAST)],
                            out_hbm.at[c, pl.ds(r0, LAST)])

    return sc_agg


def _mlp_body(h_ref, a0_ref, a1_ref, w1_ref, b1_ref, g1_ref,
              be1_ref, w2_ref, b2_ref, g2_ref, be2_ref, hout_ref):
    y = a0_ref[...] + a1_ref[...] - h_ref[...]
    h1 = jnp.maximum(
        jnp.dot(y, w1_ref[...], preferred_element_type=jnp.float32, precision=lax.Precision.HIGHEST)
        + b1_ref[...], 0.0)
    m1 = jnp.mean(h1, axis=0, keepdims=True)
    v1 = jnp.mean((h1 - m1) ** 2, axis=0, keepdims=True)
    h1 = (h1 - m1) / jnp.sqrt(v1 + 1e-5) * g1_ref[...] + be1_ref[...]
    h2 = jnp.maximum(
        jnp.dot(h1, w2_ref[...], preferred_element_type=jnp.float32, precision=lax.Precision.HIGHEST)
        + b2_ref[...], 0.0)
    m2 = jnp.mean(h2, axis=0, keepdims=True)
    v2 = jnp.mean((h2 - m2) ** 2, axis=0, keepdims=True)
    h2 = (h2 - m2) / jnp.sqrt(v2 + 1e-5) * g2_ref[...] + be2_ref[...]
    hout_ref[...] = h2


def _pool_body(G, N, b_ref, h_ref, pool_ref):
    gids = lax.broadcasted_iota(jnp.int32, (G, N), 0)
    onehot = (gids == b_ref[...]).astype(jnp.float32)
    pool_ref[...] = jnp.dot(onehot, h_ref[...],
                            preferred_element_type=jnp.float32,
                            precision=lax.Precision.HIGHEST)


def _cls_body(f_ref, gc_ref, bcn_ref, w1_ref, b1_ref, w2_ref, b2_ref, o_ref):
    f = f_ref[...]
    m = jnp.mean(f, axis=0, keepdims=True)
    v = jnp.mean((f - m) ** 2, axis=0, keepdims=True)
    f = (f - m) / jnp.sqrt(v + 1e-5) * gc_ref[...] + bcn_ref[...]
    f = jnp.maximum(
        jnp.dot(f, w1_ref[...], preferred_element_type=jnp.float32, precision=lax.Precision.HIGHEST)
        + b1_ref[...], 0.0)
    z = jnp.dot(f, w2_ref[...], preferred_element_type=jnp.float32, precision=lax.Precision.HIGHEST) + b2_ref[...]
    zm = jnp.max(z, axis=-1, keepdims=True)
    o_ref[...] = (z - zm) - jnp.log(
        jnp.sum(jnp.exp(z - zm), axis=-1, keepdims=True))


def kernel(x, edge_index, batch, W1, b1, g1, be1, W2, b2, g2, be2, gc, bcn,
           Wc1, bc1, Wc2, bc2):
    N, D = x.shape
    E = edge_index.shape[1]
    BLOCKS = W1.shape[0]
    G = 64
    C = Wc2.shape[1]

    CH = -(-E // (_NW * _L))
    CH = -(-CH // 4) * 4
    Epad = _NW * CH * _L
    NPAD = N + _L
    src = edge_index[0]
    dst = edge_index[1]
    pad = Epad - E
    lanes = jnp.arange(pad, dtype=jnp.int32) % _L
    srcp = jnp.concatenate([src, lanes]).reshape(_NW * CH, _L)
    dstp = jnp.concatenate([dst, N + lanes]).reshape(_NW * CH, _L)
    batch_row = batch.reshape(1, N)

    sc_agg = _make_sc_agg(N, D, CH, NPAD)

    mlp = pl.pallas_call(
        _mlp_body,
        out_shape=jax.ShapeDtypeStruct((N, D), jnp.float32),
    )

    pool = pl.pallas_call(
        functools.partial(_pool_body, G, N),
        out_shape=jax.ShapeDtypeStruct((G, D), jnp.float32),
    )

    cls = pl.pallas_call(
        _cls_body,
        out_shape=jax.ShapeDtypeStruct((G, C), jnp.float32),
    )

    h = x
    pooled = []
    for i in range(BLOCKS):
        acc = sc_agg(h, srcp, dstp)
        h = mlp(h, acc[0], acc[1],
                W1[i], b1[i].reshape(1, D), g1[i].reshape(1, D),
                be1[i].reshape(1, D),
                W2[i], b2[i].reshape(1, D), g2[i].reshape(1, D),
                be2[i].reshape(1, D))
        pooled.append(pool(batch_row, h))

    f = jnp.concatenate(pooled, axis=1)
    return cls(f, gc.reshape(1, -1), bcn.reshape(1, -1), Wc1,
               bc1.reshape(1, -1), Wc2, bc2.reshape(1, -1))


# speedup vs baseline: 3.8781x; 1.0398x over previous
import functools

import jax
import jax.numpy as jnp
from jax import lax
from jax.experimental import pallas as pl
from jax.experimental.pallas import tpu as pltpu
from jax.experimental.pallas import tpu_sc as plsc

_L = 128
_NW = 32
_NTILES = 16
_NBUF = 3


def _make_sc_agg(N, D, CH, NPAD):
    FULL = ((N + _NTILES - 1) // _NTILES + 7) // 8 * 8
    LAST = N - (_NTILES - 1) * FULL
    mesh = plsc.VectorSubcoreMesh(core_axis_name="c", subcore_axis_name="s")

    CH2 = CH // 2

    @functools.partial(
        pl.kernel, mesh=mesh,
        out_type=jax.ShapeDtypeStruct((2, N, D), jnp.float32),
        scratch_types=[
            pltpu.VMEM((CH2, _L), jnp.int32),
            pltpu.VMEM((CH2, _L), jnp.int32),
            pltpu.VMEM((_L, D), jnp.float32),
            pltpu.VMEM((_L, D), jnp.float32),
            pltpu.VMEM_SHARED((NPAD, D), jnp.float32),
            pltpu.SemaphoreType.DMA,
            pltpu.SemaphoreType.DMA,
            pltpu.SemaphoreType.DMA,
            pltpu.SemaphoreType.DMA,
        ],
    )
    def sc_agg(h_hbm, src_hbm, dst_hbm, out_hbm, src_v, dst_v, bufA, bufB,
               acc_sh, gA, gB, sA, sB):
        c = lax.axis_index("c")
        s = lax.axis_index("s")
        w = c * _NTILES + s
        r0 = pl.multiple_of(s * FULL, 8)

        @pl.when(s < _NTILES - 1)
        def _():
            pltpu.sync_copy(h_hbm.at[pl.ds(r0, FULL)],
                            acc_sh.at[pl.ds(r0, FULL)])

        @pl.when(s == _NTILES - 1)
        def _():
            pltpu.sync_copy(h_hbm.at[pl.ds(r0, LAST)],
                            acc_sh.at[pl.ds(r0, LAST)])

        plsc.subcore_barrier()

        for half in range(2):
            pltpu.sync_copy(src_hbm.at[pl.ds(w * CH + half * CH2, CH2)],
                            src_v)
            pltpu.sync_copy(dst_hbm.at[pl.ds(w * CH + half * CH2, CH2)],
                            dst_v)
            pltpu.async_copy(h_hbm.at[src_v.at[0]], bufA, gA)

            def body(i, carry):
                j0 = i * 2
                pltpu.make_async_copy(h_hbm.at[src_v.at[j0]], bufA, gA).wait()
                pltpu.async_copy(bufA, acc_sh.at[dst_v.at[j0]], sA, add=True)

                @pl.when(i > 0)
                def _():
                    pltpu.make_async_copy(bufB, acc_sh.at[dst_v.at[j0 - 1]],
                                          sB).wait()

                pltpu.async_copy(h_hbm.at[src_v.at[j0 + 1]], bufB, gB)
                pltpu.make_async_copy(h_hbm.at[src_v.at[j0 + 1]], bufB,
                                      gB).wait()
                pltpu.async_copy(bufB, acc_sh.at[dst_v.at[j0 + 1]], sB,
                                 add=True)

                @pl.when(j0 + 2 < CH2)
                def _():
                    pltpu.make_async_copy(bufA, acc_sh.at[dst_v.at[j0]],
                                          sA).wait()
                    pltpu.async_copy(h_hbm.at[src_v.at[j0 + 2]], bufA, gA)

                return carry

            lax.fori_loop(0, CH2 // 2, body, 0)
            pltpu.make_async_copy(bufA, acc_sh.at[dst_v.at[CH2 - 2]],
                                  sA).wait()
            pltpu.make_async_copy(bufB, acc_sh.at[dst_v.at[CH2 - 1]],
                                  sB).wait()

        plsc.subcore_barrier()

        @pl.when(s < _NTILES - 1)
        def _():
            pltpu.sync_copy(acc_sh.at[pl.ds(r0, FULL)],
                            out_hbm.at[c, pl.ds(r0, FULL)])

        @pl.when(s == _NTILES - 1)
        def _():
            pltpu.sync_copy(acc_sh.at[pl.ds(r0, LAST)],
                            out_hbm.at[c, pl.ds(r0, LAST)])

    return sc_agg


def _mlp_body(h_ref, a_ref, w1_ref, b1_ref, g1_ref,
              be1_ref, w2_ref, b2_ref, g2_ref, be2_ref, hout_ref):
    y = a_ref[0] + a_ref[1] - h_ref[...]
    h1 = jnp.maximum(
        jnp.dot(y, w1_ref[...], preferred_element_type=jnp.float32, precision=lax.Precision.HIGHEST)
        + b1_ref[...], 0.0)
    m1 = jnp.mean(h1, axis=0, keepdims=True)
    v1 = jnp.mean((h1 - m1) ** 2, axis=0, keepdims=True)
    h1 = (h1 - m1) / jnp.sqrt(v1 + 1e-5) * g1_ref[...] + be1_ref[...]
    h2 = jnp.maximum(
        jnp.dot(h1, w2_ref[...], preferred_element_type=jnp.float32, precision=lax.Precision.HIGHEST)
        + b2_ref[...], 0.0)
    m2 = jnp.mean(h2, axis=0, keepdims=True)
    v2 = jnp.mean((h2 - m2) ** 2, axis=0, keepdims=True)
    h2 = (h2 - m2) / jnp.sqrt(v2 + 1e-5) * g2_ref[...] + be2_ref[...]
    hout_ref[...] = h2


def _pool_body(G, N, b_ref, h_ref, pool_ref):
    gids = lax.broadcasted_iota(jnp.int32, (G, N), 0)
    onehot = (gids == b_ref[...]).astype(jnp.float32)
    pool_ref[...] = jnp.dot(onehot, h_ref[...],
                            preferred_element_type=jnp.float32,
                            precision=lax.Precision.HIGHEST)


def _cls_body(f_ref, gc_ref, bcn_ref, w1_ref, b1_ref, w2_ref, b2_ref, o_ref):
    f = f_ref[...]
    m = jnp.mean(f, axis=0, keepdims=True)
    v = jnp.mean((f - m) ** 2, axis=0, keepdims=True)
    f = (f - m) / jnp.sqrt(v + 1e-5) * gc_ref[...] + bcn_ref[...]
    f = jnp.maximum(
        jnp.dot(f, w1_ref[...], preferred_element_type=jnp.float32, precision=lax.Precision.HIGHEST)
        + b1_ref[...], 0.0)
    z = jnp.dot(f, w2_ref[...], preferred_element_type=jnp.float32, precision=lax.Precision.HIGHEST) + b2_ref[...]
    zm = jnp.max(z, axis=-1, keepdims=True)
    o_ref[...] = (z - zm) - jnp.log(
        jnp.sum(jnp.exp(z - zm), axis=-1, keepdims=True))


def kernel(x, edge_index, batch, W1, b1, g1, be1, W2, b2, g2, be2, gc, bcn,
           Wc1, bc1, Wc2, bc2):
    N, D = x.shape
    E = edge_index.shape[1]
    BLOCKS = W1.shape[0]
    G = 64
    C = Wc2.shape[1]

    CH = -(-E // (_NW * _L))
    CH = -(-CH // 4) * 4
    Epad = _NW * CH * _L
    NPAD = N + _L
    src = edge_index[0]
    dst = edge_index[1]
    pad = Epad - E
    lanes = jnp.arange(pad, dtype=jnp.int32) % _L
    srcp = jnp.concatenate([src, lanes]).reshape(_NW * CH, _L)
    dstp = jnp.concatenate([dst, N + lanes]).reshape(_NW * CH, _L)
    batch_row = batch.reshape(1, N)

    sc_agg = _make_sc_agg(N, D, CH, NPAD)

    mlp = pl.pallas_call(
        _mlp_body,
        out_shape=jax.ShapeDtypeStruct((N, D), jnp.float32),
    )

    pool = pl.pallas_call(
        functools.partial(_pool_body, G, N),
        out_shape=jax.ShapeDtypeStruct((G, D), jnp.float32),
    )

    cls = pl.pallas_call(
        _cls_body,
        out_shape=jax.ShapeDtypeStruct((G, C), jnp.float32),
    )

    h = x
    pooled = []
    for i in range(BLOCKS):
        acc = sc_agg(h, srcp, dstp)
        h = mlp(h, acc,
                W1[i], b1[i].reshape(1, D), g1[i].reshape(1, D),
                be1[i].reshape(1, D),
                W2[i], b2[i].reshape(1, D), g2[i].reshape(1, D),
                be2[i].reshape(1, D))
        pooled.append(pool(batch_row, h))

    f = jnp.concatenate(pooled, axis=1)
    return cls(f, gc.reshape(1, -1), bcn.reshape(1, -1), Wc1,
               bc1.reshape(1, -1), Wc2, bc2.reshape(1, -1))
